# baseline (speedup 1.0000x reference)
import functools

import jax
import jax.numpy as jnp
from jax import lax
from jax.experimental import pallas as pl
from jax.experimental.pallas import tpu as pltpu
from jax.experimental.pallas import tpu_sc as plsc

NUM_WORKERS = 32
CHUNK = 64
NBUF = 2
PRE = 2


@functools.cache
def _build(n_total, n_chunks, embed_dim, n_rows):
    mesh = plsc.VectorSubcoreMesh(core_axis_name="c", subcore_axis_name="s")
    n_sub = 16
    rows_per_sub = (n_rows - 1) // n_sub
    rem_base = rows_per_sub * n_sub
    rem = n_rows - rem_base

    @functools.partial(
        pl.kernel,
        mesh=mesh,
        out_type=jax.ShapeDtypeStruct((n_total, embed_dim), jnp.float32),
        scratch_types=[
            pltpu.VMEM((n_chunks, CHUNK), jnp.int32),
            pltpu.VMEM((NBUF, CHUNK, embed_dim), jnp.float32),
            pltpu.VMEM_SHARED((n_rows, embed_dim), jnp.float32),
            pltpu.SemaphoreType.DMA,
            pltpu.SemaphoreType.DMA,
            pltpu.SemaphoreType.DMA,
        ],
    )
    def gather_kernel(idx_hbm, table_hbm, out_hbm, idx_v, rows_v, table_sp,
                      gsem, osem, ssem):
        sid = lax.axis_index("s")
        wid = sid * 2 + lax.axis_index("c")
        base = wid * (n_chunks * CHUNK)
        pltpu.sync_copy(idx_hbm.at[wid], idx_v)

        stage = pltpu.async_copy(
            table_hbm.at[pl.ds(sid * rows_per_sub, rows_per_sub)],
            table_sp.at[pl.ds(sid * rows_per_sub, rows_per_sub)],
            ssem,
        )

        @pl.when(sid == 0)
        def _():
            pltpu.async_copy(
                table_hbm.at[pl.ds(rem_base, rem)],
                table_sp.at[pl.ds(rem_base, rem)],
                ssem,
            )

        def start_gather(src, buf, chunk):
            pltpu.async_copy(src.at[idx_v.at[chunk]], rows_v.at[buf], gsem)

        def wait_gather(src, buf, chunk):
            pltpu.make_async_copy(
                src.at[idx_v.at[chunk]], rows_v.at[buf], gsem
            ).wait()

        def start_out(buf, chunk):
            pltpu.async_copy(
                rows_v.at[buf],
                out_hbm.at[pl.ds(base + chunk * CHUNK, CHUNK)],
                osem,
            )

        def wait_out(buf):
            pltpu.make_async_copy(
                rows_v.at[buf], out_hbm.at[pl.ds(base, CHUNK)], osem
            ).wait()

        for b in range(NBUF):
            start_gather(table_hbm, b, b)
        for j in range(PRE // NBUF):
            first = j * NBUF
            for b in range(NBUF):
                wait_gather(table_hbm, b, first + b)
                start_out(b, first + b)
            for b in range(NBUF):
                wait_out(b)
                nxt = first + NBUF + b
                if nxt < PRE:
                    start_gather(table_hbm, b, nxt)

        stage.wait()

        @pl.when(sid == 0)
        def _():
            pltpu.make_async_copy(
                table_hbm.at[pl.ds(rem_base, rem)],
                table_sp.at[pl.ds(rem_base, rem)],
                ssem,
            ).wait()

        plsc.subcore_barrier()

        for b in range(NBUF):
            start_gather(table_sp, b, PRE + b)

        def body(j, carry):
            first = PRE + j * NBUF
            for b in range(NBUF):
                wait_gather(table_sp, b, first + b)
                start_out(b, first + b)
            for b in range(NBUF):
                wait_out(b)
                nxt = first + NBUF + b

                @pl.when(nxt < n_chunks)
                def _():
                    start_gather(table_sp, b, nxt)

            return carry

        lax.fori_loop(0, (n_chunks - PRE) // NBUF, body, 0)

    return gather_kernel


def kernel(position_ids, table):
    n_total = position_ids.size
    embed_dim = table.shape[1]
    n_chunks = n_total // (NUM_WORKERS * CHUNK)
    idx3 = position_ids.reshape(NUM_WORKERS, n_chunks, CHUNK)
    out = _build(n_total, n_chunks, embed_dim, table.shape[0])(idx3, table)
    return out.reshape(position_ids.shape + (embed_dim,))


# speedup vs baseline: 1.0492x; 1.0492x over previous
import functools

import jax
import jax.numpy as jnp
from jax import lax
from jax.experimental import pallas as pl
from jax.experimental.pallas import tpu as pltpu
from jax.experimental.pallas import tpu_sc as plsc

NUM_WORKERS = 32
CHUNK = 128
NBUF = 2
PRE = 2


@functools.cache
def _build(n_total, n_chunks, embed_dim, n_rows):
    mesh = plsc.VectorSubcoreMesh(core_axis_name="c", subcore_axis_name="s")
    n_sub = 16
    rows_per_sub = (n_rows - 1) // n_sub
    rem_base = rows_per_sub * n_sub
    rem = n_rows - rem_base

    @functools.partial(
        pl.kernel,
        mesh=mesh,
        out_type=jax.ShapeDtypeStruct((n_total, embed_dim), jnp.float32),
        scratch_types=[
            pltpu.VMEM((n_chunks, CHUNK), jnp.int32),
            pltpu.VMEM((NBUF, CHUNK, embed_dim), jnp.float32),
            pltpu.VMEM_SHARED((n_rows, embed_dim), jnp.float32),
            pltpu.SemaphoreType.DMA,
            pltpu.SemaphoreType.DMA,
            pltpu.SemaphoreType.DMA,
        ],
    )
    def gather_kernel(idx_hbm, table_hbm, out_hbm, idx_v, rows_v, table_sp,
                      gsem, osem, ssem):
        sid = lax.axis_index("s")
        wid = sid * 2 + lax.axis_index("c")
        base = wid * (n_chunks * CHUNK)
        pltpu.sync_copy(idx_hbm.at[wid], idx_v)

        stage = pltpu.async_copy(
            table_hbm.at[pl.ds(sid * rows_per_sub, rows_per_sub)],
            table_sp.at[pl.ds(sid * rows_per_sub, rows_per_sub)],
            ssem,
        )

        @pl.when(sid == 0)
        def _():
            pltpu.async_copy(
                table_hbm.at[pl.ds(rem_base, rem)],
                table_sp.at[pl.ds(rem_base, rem)],
                ssem,
            )

        def start_gather(src, buf, chunk):
            pltpu.async_copy(src.at[idx_v.at[chunk]], rows_v.at[buf], gsem)

        def wait_gather(src, buf, chunk):
            pltpu.make_async_copy(
                src.at[idx_v.at[chunk]], rows_v.at[buf], gsem
            ).wait()

        def start_out(buf, chunk):
            pltpu.async_copy(
                rows_v.at[buf],
                out_hbm.at[pl.ds(base + chunk * CHUNK, CHUNK)],
                osem,
            )

        def wait_out(buf):
            pltpu.make_async_copy(
                rows_v.at[buf], out_hbm.at[pl.ds(base, CHUNK)], osem
            ).wait()

        for b in range(NBUF):
            start_gather(table_hbm, b, b)
        for j in range(PRE // NBUF):
            first = j * NBUF
            for b in range(NBUF):
                wait_gather(table_hbm, b, first + b)
                start_out(b, first + b)
            for b in range(NBUF):
                wait_out(b)
                nxt = first + NBUF + b
                if nxt < PRE:
                    start_gather(table_hbm, b, nxt)

        stage.wait()

        @pl.when(sid == 0)
        def _():
            pltpu.make_async_copy(
                table_hbm.at[pl.ds(rem_base, rem)],
                table_sp.at[pl.ds(rem_base, rem)],
                ssem,
            ).wait()

        plsc.subcore_barrier()

        for b in range(NBUF):
            start_gather(table_sp, b, PRE + b)

        def body(j, carry):
            first = PRE + j * NBUF
            for b in range(NBUF):
                wait_gather(table_sp, b, first + b)
                start_out(b, first + b)
            for b in range(NBUF):
                wait_out(b)
                nxt = first + NBUF + b

                @pl.when(nxt < n_chunks)
                def _():
                    start_gather(table_sp, b, nxt)

            return carry

        lax.fori_loop(0, (n_chunks - PRE) // NBUF, body, 0)

    return gather_kernel


def kernel(position_ids, table):
    n_total = position_ids.size
    embed_dim = table.shape[1]
    n_chunks = n_total // (NUM_WORKERS * CHUNK)
    idx3 = position_ids.reshape(NUM_WORKERS, n_chunks, CHUNK)
    out = _build(n_total, n_chunks, embed_dim, table.shape[0])(idx3, table)
    return out.reshape(position_ids.shape + (embed_dim,))


# speedup vs baseline: 1.0531x; 1.0036x over previous
import functools

import jax
import jax.numpy as jnp
from jax import lax
from jax.experimental import pallas as pl
from jax.experimental.pallas import tpu as pltpu
from jax.experimental.pallas import tpu_sc as plsc

NUM_WORKERS = 32
CHUNK = 128
NBUF = 2
PRE = 2


@functools.cache
def _build(n_total, n_chunks, embed_dim, n_rows):
    mesh = plsc.VectorSubcoreMesh(core_axis_name="c", subcore_axis_name="s")
    n_sub = 16
    rows_per_sub = (n_rows - 1) // n_sub
    rem_base = rows_per_sub * n_sub
    rem = n_rows - rem_base

    @functools.partial(
        pl.kernel,
        mesh=mesh,
        out_type=jax.ShapeDtypeStruct((n_total, embed_dim), jnp.float32),
        scratch_types=[
            pltpu.VMEM((n_chunks, CHUNK), jnp.int32),
            pltpu.VMEM((NBUF, CHUNK, embed_dim), jnp.float32),
            pltpu.VMEM_SHARED((n_rows, embed_dim), jnp.float32),
            pltpu.SemaphoreType.DMA,
            pltpu.SemaphoreType.DMA,
            pltpu.SemaphoreType.DMA,
            pltpu.SemaphoreType.DMA,
        ],
    )
    def gather_kernel(idx_hbm, table_hbm, out_hbm, idx_v, rows_v, table_sp,
                      gsem, osem, ssem, isem):
        sid = lax.axis_index("s")
        wid = sid * 2 + lax.axis_index("c")
        base = wid * (n_chunks * CHUNK)
        head = 8
        pltpu.sync_copy(idx_hbm.at[wid, pl.ds(0, head)], idx_v.at[pl.ds(0, head)])
        idx_rest = pltpu.async_copy(
            idx_hbm.at[wid, pl.ds(head, n_chunks - head)],
            idx_v.at[pl.ds(head, n_chunks - head)],
            isem,
        )

        stage = pltpu.async_copy(
            table_hbm.at[pl.ds(sid * rows_per_sub, rows_per_sub)],
            table_sp.at[pl.ds(sid * rows_per_sub, rows_per_sub)],
            ssem,
        )

        @pl.when(sid == 0)
        def _():
            pltpu.async_copy(
                table_hbm.at[pl.ds(rem_base, rem)],
                table_sp.at[pl.ds(rem_base, rem)],
                ssem,
            )

        def start_gather(src, buf, chunk):
            pltpu.async_copy(src.at[idx_v.at[chunk]], rows_v.at[buf], gsem)

        def wait_gather(src, buf, chunk):
            pltpu.make_async_copy(
                src.at[idx_v.at[chunk]], rows_v.at[buf], gsem
            ).wait()

        def start_out(buf, chunk):
            pltpu.async_copy(
                rows_v.at[buf],
                out_hbm.at[pl.ds(base + chunk * CHUNK, CHUNK)],
                osem,
            )

        def wait_out(buf):
            pltpu.make_async_copy(
                rows_v.at[buf], out_hbm.at[pl.ds(base, CHUNK)], osem
            ).wait()

        for b in range(NBUF):
            start_gather(table_hbm, b, b)
        for j in range(PRE // NBUF):
            first = j * NBUF
            for b in range(NBUF):
                wait_gather(table_hbm, b, first + b)
                start_out(b, first + b)
            for b in range(NBUF):
                wait_out(b)
                nxt = first + NBUF + b
                if nxt < PRE:
                    start_gather(table_hbm, b, nxt)

        stage.wait()
        idx_rest.wait()

        @pl.when(sid == 0)
        def _():
            pltpu.make_async_copy(
                table_hbm.at[pl.ds(rem_base, rem)],
                table_sp.at[pl.ds(rem_base, rem)],
                ssem,
            ).wait()

        plsc.subcore_barrier()

        for b in range(NBUF):
            start_gather(table_sp, b, PRE + b)

        def body(j, carry):
            first = PRE + j * NBUF
            for b in range(NBUF):
                wait_gather(table_sp, b, first + b)
                start_out(b, first + b)
            for b in range(NBUF):
                wait_out(b)
                nxt = first + NBUF + b

                @pl.when(nxt < n_chunks)
                def _():
                    start_gather(table_sp, b, nxt)

            return carry

        lax.fori_loop(0, (n_chunks - PRE) // NBUF, body, 0)

    return gather_kernel


def kernel(position_ids, table):
    n_total = position_ids.size
    embed_dim = table.shape[1]
    n_chunks = n_total // (NUM_WORKERS * CHUNK)
    idx3 = position_ids.reshape(NUM_WORKERS, n_chunks, CHUNK)
    out = _build(n_total, n_chunks, embed_dim, table.shape[0])(idx3, table)
    return out.reshape(position_ids.shape + (embed_dim,))
